# baseline (speedup 1.0000x reference)
import functools

import jax
import jax.numpy as jnp
from jax import lax
from jax.experimental import pallas as pl
from jax.experimental.pallas import tpu as pltpu
from jax.experimental.pallas import tpu_sc as plsc

DIM = 768
N_EXPERTS = 64
TOPK = 8
N_GROUPS = 8
GROUP_SIZE = 8
N_TOPK_GROUP = 4
ROUTED_SCALE = 2.5
T_TOTAL = 32768
NW = 32
N_PHASES = 4
T_PHASE = T_TOTAL // N_PHASES
CW = T_PHASE // NW
LANES = 16

_SORT8 = ((0, 1), (2, 3), (4, 5), (6, 7),
          (0, 2), (1, 3), (4, 6), (5, 7),
          (1, 2), (5, 6),
          (0, 4), (1, 5), (2, 6), (3, 7),
          (2, 4), (3, 5),
          (1, 2), (3, 4), (5, 6))
_BMERGE8 = ((0, 4), (1, 5), (2, 6), (3, 7),
            (0, 2), (1, 3), (4, 6), (5, 7),
            (0, 1), (2, 3), (4, 5), (6, 7))


def _scores_tc_kernel(x_ref, w_ref, b_ref, out_ref):
    z = lax.dot_general(w_ref[...], x_ref[...],
                        (((1,), (1,)), ((), ())),
                        preferred_element_type=jnp.float32)
    out_ref[...] = jax.nn.sigmoid(z) + b_ref[...]


def _ce(vals, idxs, a, b):
    c = vals[a] >= vals[b]
    hi = jnp.maximum(vals[a], vals[b])
    lo = jnp.minimum(vals[a], vals[b])
    ia = jnp.where(c, idxs[a], idxs[b])
    ib = jnp.where(c, idxs[b], idxs[a])
    vals[a], vals[b], idxs[a], idxs[b] = hi, lo, ia, ib


def _route_sc_kernel(scores_hbm, w_hbm, i_hbm, sv_ref, wv_ref, iv_ref):
    cid = lax.axis_index("c")
    sid = lax.axis_index("s")
    wid = sid * 2 + cid
    pltpu.sync_copy(scores_hbm.at[pl.ds(wid * N_EXPERTS, N_EXPERTS)], sv_ref)

    neg = jnp.full((LANES,), -jnp.inf, jnp.float32)
    onei = jnp.full((LANES,), 1, jnp.int32)
    zeroi = jnp.full((LANES,), 0, jnp.int32)

    def step(t, carry):
        base = t * LANES

        gs = []
        for g in range(N_GROUPS):
            e0 = g * GROUP_SIZE
            v0 = sv_ref[e0, pl.ds(base, LANES)]
            v1 = sv_ref[e0 + 1, pl.ds(base, LANES)]
            m1 = jnp.maximum(v0, v1)
            m2 = jnp.minimum(v0, v1)
            for e in range(e0 + 2, e0 + GROUP_SIZE):
                v = sv_ref[e, pl.ds(base, LANES)]
                hi = jnp.maximum(m1, v)
                m2 = jnp.maximum(m2, jnp.minimum(m1, v))
                m1 = hi
            gs.append(m1 + m2)

        rank = [zeroi] * N_GROUPS
        for j in range(N_GROUPS):
            for g in range(j + 1, N_GROUPS):
                c = gs[j] >= gs[g]
                rank[g] = rank[g] + jnp.where(c, onei, zeroi)
                rank[j] = rank[j] + jnp.where(c, zeroi, onei)
        sel = [rank[g] < N_TOPK_GROUP for g in range(N_GROUPS)]

        rv = ri = None
        for g in range(N_GROUPS):
            e0 = g * GROUP_SIZE
            sv = [sv_ref[e0 + i, pl.ds(base, LANES)] for i in range(GROUP_SIZE)]
            si = [jnp.full((LANES,), e0 + i, jnp.int32) for i in range(GROUP_SIZE)]
            for (a, b) in _SORT8:
                _ce(sv, si, a, b)
            sv = [jnp.where(sel[g], val, neg) for val in sv]
            if rv is None:
                rv, ri = sv, si
            else:
                mv, mi = [], []
                for k in range(TOPK):
                    c = rv[k] >= sv[TOPK - 1 - k]
                    mv.append(jnp.maximum(rv[k], sv[TOPK - 1 - k]))
                    mi.append(jnp.where(c, ri[k], si[TOPK - 1 - k]))
                for (a, b) in _BMERGE8:
                    _ce(mv, mi, a, b)
                rv, ri = mv, mi

        total = rv[0]
        for k in range(1, TOPK):
            total = total + rv[k]
        inv = ROUTED_SCALE / total
        for k in range(TOPK):
            wv_ref[0, k, pl.ds(base, LANES)] = rv[k] * inv
            iv_ref[0, k, pl.ds(base, LANES)] = ri[k]
        return carry

    lax.fori_loop(0, CW // LANES, step, 0, unroll=False)

    pltpu.sync_copy(wv_ref, w_hbm.at[pl.ds(wid, 1)])
    pltpu.sync_copy(iv_ref, i_hbm.at[pl.ds(wid, 1)])


@jax.jit
def kernel(x, W, b):
    b2 = b.reshape(N_EXPERTS, 1)
    mesh = plsc.VectorSubcoreMesh(core_axis_name="c", subcore_axis_name="s")
    route = pl.kernel(
        _route_sc_kernel,
        out_type=(
            jax.ShapeDtypeStruct((NW, TOPK, CW), jnp.float32),
            jax.ShapeDtypeStruct((NW, TOPK, CW), jnp.int32),
        ),
        mesh=mesh,
        scratch_types=[
            pltpu.VMEM((N_EXPERTS, CW), jnp.float32),
            pltpu.VMEM((1, TOPK, CW), jnp.float32),
            pltpu.VMEM((1, TOPK, CW), jnp.int32),
        ],
    )
    scores_tc = pl.pallas_call(
        _scores_tc_kernel,
        grid=(NW,),
        in_specs=[
            pl.BlockSpec((CW, DIM), lambda j: (j, 0)),
            pl.BlockSpec((N_EXPERTS, DIM), lambda j: (0, 0)),
            pl.BlockSpec((N_EXPERTS, 1), lambda j: (0, 0)),
        ],
        out_specs=pl.BlockSpec((N_EXPERTS, CW), lambda j: (j, 0)),
        out_shape=jax.ShapeDtypeStruct((NW * N_EXPERTS, CW), jnp.float32),
    )

    weights_parts, indices_parts = [], []
    for p in range(N_PHASES):
        xp = lax.slice_in_dim(x, p * T_PHASE, (p + 1) * T_PHASE, axis=0)
        scores_p = scores_tc(xp, W, b2)
        wp, ip = route(scores_p)
        weights_parts.append(jnp.transpose(wp, (0, 2, 1)).reshape(T_PHASE, TOPK))
        indices_parts.append(jnp.transpose(ip, (0, 2, 1)).reshape(T_PHASE, TOPK))
    return (jnp.concatenate(weights_parts, axis=0),
            jnp.concatenate(indices_parts, axis=0))


# speedup vs baseline: 2.2723x; 2.2723x over previous
import jax
import jax.numpy as jnp
from jax import lax
from jax.experimental import pallas as pl
from jax.experimental.pallas import tpu as pltpu
from jax.experimental.pallas import tpu_sc as plsc

DIM = 768
N_EXPERTS = 64
TOPK = 8
N_GROUPS = 8
GROUP_SIZE = 8
N_TOPK_GROUP = 4
ROUTED_SCALE = 2.5
T_TOTAL = 32768
NW = 32
CW = T_TOTAL // NW
LANES = 16

_SORT8 = ((0, 1), (2, 3), (4, 5), (6, 7),
          (0, 2), (1, 3), (4, 6), (5, 7),
          (1, 2), (5, 6),
          (0, 4), (1, 5), (2, 6), (3, 7),
          (2, 4), (3, 5),
          (1, 2), (3, 4), (5, 6))
_BMERGE8 = ((0, 4), (1, 5), (2, 6), (3, 7),
            (0, 2), (1, 3), (4, 6), (5, 7),
            (0, 1), (2, 3), (4, 5), (6, 7))

_I31 = 0x7FFFFFFF


def _scores_tc_kernel(x_ref, w_ref, b_ref, out_ref):
    z = lax.dot_general(w_ref[...], x_ref[...],
                        (((1,), (1,)), ((), ())),
                        preferred_element_type=jnp.float32)
    out_ref[...] = jax.nn.sigmoid(z) + b_ref[...]


def _encode(v, ident):
    u = lax.bitcast_convert_type(v, jnp.int32)
    sk = u ^ (lax.shift_right_arithmetic(u, 31) & _I31)
    return (sk & ~63) | (63 - ident)


def _decode_val(key):
    sk = key & ~63
    return lax.bitcast_convert_type(
        sk ^ (lax.shift_right_arithmetic(sk, 31) & _I31), jnp.float32)


def _decode_id(key):
    return 63 - (key & 63)


def _ce(keys, a, b):
    hi = jnp.maximum(keys[a], keys[b])
    lo = jnp.minimum(keys[a], keys[b])
    keys[a], keys[b] = hi, lo


def _route_sc_kernel(scores_hbm, w_hbm, i_hbm, sv_ref, wv_ref, iv_ref):
    cid = lax.axis_index("c")
    sid = lax.axis_index("s")
    wid = sid * 2 + cid
    pltpu.sync_copy(scores_hbm.at[pl.ds(wid * N_EXPERTS, N_EXPERTS)], sv_ref)

    intmin = jnp.full((LANES,), jnp.iinfo(jnp.int32).min, jnp.int32)

    def step(t, carry):
        base = t * LANES

        gs = []
        for g in range(N_GROUPS):
            e0 = g * GROUP_SIZE
            ks = [_encode(sv_ref[e0 + i, pl.ds(base, LANES)],
                          jnp.full((LANES,), e0 + i, jnp.int32))
                  for i in range(GROUP_SIZE)]
            for (a, b) in _SORT8:
                _ce(ks, a, b)
            gs.append(_decode_val(ks[0]) + _decode_val(ks[1]))
            for i in range(GROUP_SIZE):
                sv_ref[e0 + i, pl.ds(base, LANES)] = lax.bitcast_convert_type(
                    ks[i], jnp.float32)

        gk = [_encode(gs[g], jnp.full((LANES,), g, jnp.int32))
              for g in range(N_GROUPS)]
        gsorted = list(gk)
        for (a, b) in _SORT8:
            _ce(gsorted, a, b)
        thresh = gsorted[N_TOPK_GROUP - 1]
        sel = [gk[g] >= thresh for g in range(N_GROUPS)]

        rk = None
        for g in range(N_GROUPS):
            e0 = g * GROUP_SIZE
            ks = [lax.bitcast_convert_type(
                      sv_ref[e0 + i, pl.ds(base, LANES)], jnp.int32)
                  for i in range(GROUP_SIZE)]
            ks = [jnp.where(sel[g], k, intmin) for k in ks]
            if rk is None:
                rk = ks
            else:
                mk = [jnp.maximum(rk[k], ks[TOPK - 1 - k]) for k in range(TOPK)]
                for (a, b) in _BMERGE8:
                    _ce(mk, a, b)
                rk = mk

        vals = [_decode_val(k) for k in rk]
        total = vals[0]
        for k in range(1, TOPK):
            total = total + vals[k]
        inv = ROUTED_SCALE / total
        for k in range(TOPK):
            wv_ref[0, k, pl.ds(base, LANES)] = vals[k] * inv
            iv_ref[0, k, pl.ds(base, LANES)] = _decode_id(rk[k])
        return carry

    lax.fori_loop(0, CW // LANES, step, 0, unroll=False)

    pltpu.sync_copy(wv_ref, w_hbm.at[pl.ds(wid, 1)])
    pltpu.sync_copy(iv_ref, i_hbm.at[pl.ds(wid, 1)])


@jax.jit
def kernel(x, W, b):
    scores = pl.pallas_call(
        _scores_tc_kernel,
        grid=(NW,),
        in_specs=[
            pl.BlockSpec((CW, DIM), lambda j: (j, 0)),
            pl.BlockSpec((N_EXPERTS, DIM), lambda j: (0, 0)),
            pl.BlockSpec((N_EXPERTS, 1), lambda j: (0, 0)),
        ],
        out_specs=pl.BlockSpec((N_EXPERTS, CW), lambda j: (j, 0)),
        out_shape=jax.ShapeDtypeStruct((NW * N_EXPERTS, CW), jnp.float32),
    )(x, W, b.reshape(N_EXPERTS, 1))

    mesh = plsc.VectorSubcoreMesh(core_axis_name="c", subcore_axis_name="s")
    route = pl.kernel(
        _route_sc_kernel,
        out_type=(
            jax.ShapeDtypeStruct((NW, TOPK, CW), jnp.float32),
            jax.ShapeDtypeStruct((NW, TOPK, CW), jnp.int32),
        ),
        mesh=mesh,
        scratch_types=[
            pltpu.VMEM((N_EXPERTS, CW), jnp.float32),
            pltpu.VMEM((1, TOPK, CW), jnp.float32),
            pltpu.VMEM((1, TOPK, CW), jnp.int32),
        ],
    )
    weights, indices = route(scores)
    weights = jnp.transpose(weights, (0, 2, 1)).reshape(T_TOTAL, TOPK)
    indices = jnp.transpose(indices, (0, 2, 1)).reshape(T_TOTAL, TOPK)
    return weights, indices


# speedup vs baseline: 2.4307x; 1.0697x over previous
import jax
import jax.numpy as jnp
from jax import lax
from jax.experimental import pallas as pl
from jax.experimental.pallas import tpu as pltpu
from jax.experimental.pallas import tpu_sc as plsc

DIM = 768
N_EXPERTS = 64
TOPK = 8
N_GROUPS = 8
GROUP_SIZE = 8
N_TOPK_GROUP = 4
ROUTED_SCALE = 2.5
T_TOTAL = 32768
NW = 32
CW = T_TOTAL // NW
LANES = 16
TBLK = 4096
WPB = TBLK // CW

_SORT8 = ((0, 1), (2, 3), (4, 5), (6, 7),
          (0, 2), (1, 3), (4, 6), (5, 7),
          (1, 2), (5, 6),
          (0, 4), (1, 5), (2, 6), (3, 7),
          (2, 4), (3, 5),
          (1, 2), (3, 4), (5, 6))
_BMERGE8 = ((0, 4), (1, 5), (2, 6), (3, 7),
            (0, 2), (1, 3), (4, 6), (5, 7),
            (0, 1), (2, 3), (4, 5), (6, 7))


def _scores_tc_kernel(x_ref, w_ref, b_ref, out_ref):
    z = lax.dot_general(w_ref[...], x_ref[...],
                        (((1,), (1,)), ((), ())),
                        preferred_element_type=jnp.float32)
    out_ref[...] = jax.nn.sigmoid(z) + b_ref[...]


def _ce(vals, idxs, a, b):
    c = vals[a] >= vals[b]
    hi = jnp.maximum(vals[a], vals[b])
    lo = jnp.minimum(vals[a], vals[b])
    ia = jnp.where(c, idxs[a], idxs[b])
    ib = jnp.where(c, idxs[b], idxs[a])
    vals[a], vals[b], idxs[a], idxs[b] = hi, lo, ia, ib


def _route_sc_kernel(scores_hbm, w_hbm, i_hbm, sv_ref, wp_ref, ip_ref):
    cid = lax.axis_index("c")
    sid = lax.axis_index("s")
    wid = sid * 2 + cid
    r0 = (wid // WPB) * N_EXPERTS
    c0 = (wid % WPB) * CW
    pltpu.sync_copy(scores_hbm.at[pl.ds(r0, N_EXPERTS), pl.ds(c0, CW)], sv_ref)

    neg = jnp.full((LANES,), -jnp.inf, jnp.float32)
    onei = jnp.full((LANES,), 1, jnp.int32)
    zeroi = jnp.full((LANES,), 0, jnp.int32)
    def step(t, carry):
        base = t * LANES

        gs = []
        for g in range(N_GROUPS):
            e0 = g * GROUP_SIZE
            v0 = sv_ref[e0, pl.ds(base, LANES)]
            v1 = sv_ref[e0 + 1, pl.ds(base, LANES)]
            m1 = jnp.maximum(v0, v1)
            m2 = jnp.minimum(v0, v1)
            for e in range(e0 + 2, e0 + GROUP_SIZE):
                v = sv_ref[e, pl.ds(base, LANES)]
                hi = jnp.maximum(m1, v)
                m2 = jnp.maximum(m2, jnp.minimum(m1, v))
                m1 = hi
            gs.append(m1 + m2)

        rank = [zeroi] * N_GROUPS
        for j in range(N_GROUPS):
            for g in range(j + 1, N_GROUPS):
                c = gs[j] >= gs[g]
                rank[g] = rank[g] + jnp.where(c, onei, zeroi)
                rank[j] = rank[j] + jnp.where(c, zeroi, onei)
        sel = [rank[g] < N_TOPK_GROUP for g in range(N_GROUPS)]

        rv = ri = None
        for g in range(N_GROUPS):
            e0 = g * GROUP_SIZE
            sv = [sv_ref[e0 + i, pl.ds(base, LANES)] for i in range(GROUP_SIZE)]
            si = [jnp.full((LANES,), e0 + i, jnp.int32) for i in range(GROUP_SIZE)]
            for (a, b) in _SORT8:
                _ce(sv, si, a, b)
            sv = [jnp.where(sel[g], val, neg) for val in sv]
            if rv is None:
                rv, ri = sv, si
            else:
                mv, mi = [], []
                for k in range(TOPK):
                    c = rv[k] >= sv[TOPK - 1 - k]
                    mv.append(jnp.maximum(rv[k], sv[TOPK - 1 - k]))
                    mi.append(jnp.where(c, ri[k], si[TOPK - 1 - k]))
                for (a, b) in _BMERGE8:
                    _ce(mv, mi, a, b)
                rv, ri = mv, mi

        total = rv[0]
        for k in range(1, TOPK):
            total = total + rv[k]
        inv = ROUTED_SCALE / total
        for k in range(TOPK):
            wp_ref[0, k, pl.ds(base, LANES)] = rv[k] * inv
            ip_ref[0, k, pl.ds(base, LANES)] = ri[k]
        return carry

    lax.fori_loop(0, CW // LANES, step, 0, unroll=False)

    pltpu.sync_copy(wp_ref, w_hbm.at[pl.ds(wid, 1)])
    pltpu.sync_copy(ip_ref, i_hbm.at[pl.ds(wid, 1)])


@jax.jit
def kernel(x, W, b):
    grid = T_TOTAL // TBLK
    scores = pl.pallas_call(
        _scores_tc_kernel,
        grid=(grid,),
        in_specs=[
            pl.BlockSpec((TBLK, DIM), lambda j: (j, 0)),
            pl.BlockSpec((N_EXPERTS, DIM), lambda j: (0, 0)),
            pl.BlockSpec((N_EXPERTS, 1), lambda j: (0, 0)),
        ],
        out_specs=pl.BlockSpec((N_EXPERTS, TBLK), lambda j: (j, 0)),
        out_shape=jax.ShapeDtypeStruct((grid * N_EXPERTS, TBLK), jnp.float32),
    )(x, W, b.reshape(N_EXPERTS, 1))

    mesh = plsc.VectorSubcoreMesh(core_axis_name="c", subcore_axis_name="s")
    route = pl.kernel(
        _route_sc_kernel,
        out_type=(
            jax.ShapeDtypeStruct((NW, TOPK, CW), jnp.float32),
            jax.ShapeDtypeStruct((NW, TOPK, CW), jnp.int32),
        ),
        mesh=mesh,
        scratch_types=[
            pltpu.VMEM((N_EXPERTS, CW), jnp.float32),
            pltpu.VMEM((1, TOPK, CW), jnp.float32),
            pltpu.VMEM((1, TOPK, CW), jnp.int32),
        ],
    )
    weights, indices = route(scores)
    weights = jnp.transpose(weights, (0, 2, 1)).reshape(T_TOTAL, TOPK)
    indices = jnp.transpose(indices, (0, 2, 1)).reshape(T_TOTAL, TOPK)
    return weights, indices
